# baseline (speedup 1.0000x reference)
import jax
import jax.numpy as jnp
from jax import lax
from jax.experimental import pallas as pl
from jax.experimental.pallas import tpu as pltpu
from jax.experimental.pallas import tpu_sc as plsc

B, L, D = 4096, 200, 128
NC, NS, LANES = 2, 16, 16
NW = NC * NS
RPW = B // NW
NCH = (L + LANES - 1) // LANES

BBM = 128
NSTEPS = B // BBM
NSLOT = 4


def _sc_body(ids_hbm, lab_hbm, mask_hbm, ids_v, lab_v, mask_v, sem):
    wid = lax.axis_index("s") * NC + lax.axis_index("c")
    base = wid * RPW * L

    pltpu.make_async_copy(
        ids_hbm.at[pl.ds(base, RPW * L)],
        ids_v.at[pl.ds(0, RPW * L)], sem).start()
    pltpu.make_async_copy(
        ids_hbm.at[pl.ds(base, RPW * L)],
        ids_v.at[pl.ds(0, RPW * L)], sem).wait()

    lane = lax.iota(jnp.int32, LANES)
    last_valid = L - 1 - 16 * (NCH - 1)

    def row_body(r, _):
        off = r * L
        for k in range(NCH):
            lab = ids_v[pl.ds(off + 16 * k + 1, 16)]
            if k == NCH - 1:
                lab = jnp.where(lane < last_valid, lab, 0)
            lab_v[pl.ds(off + 16 * k, 16)] = lab
            mask_v[pl.ds(off + 16 * k, 16)] = jnp.where(lab != 0, 1, 0)
        return 0

    lax.fori_loop(0, RPW, row_body, 0)

    pltpu.make_async_copy(
        lab_v.at[pl.ds(0, RPW * L)], lab_hbm.at[pl.ds(base, RPW * L)],
        sem).start()
    pltpu.make_async_copy(
        mask_v.at[pl.ds(0, RPW * L)], mask_hbm.at[pl.ds(base, RPW * L)],
        sem).start()
    pltpu.make_async_copy(
        lab_v.at[pl.ds(0, RPW * L)], lab_hbm.at[pl.ds(base, RPW * L)],
        sem).wait()
    pltpu.make_async_copy(
        mask_v.at[pl.ds(0, RPW * L)], mask_hbm.at[pl.ds(base, RPW * L)],
        sem).wait()


def _sc_labels_mask(ids_flat):
    mesh = plsc.VectorSubcoreMesh(core_axis_name="c", subcore_axis_name="s")
    f = pl.kernel(
        _sc_body,
        out_type=[
            jax.ShapeDtypeStruct((B * L,), jnp.int32),
            jax.ShapeDtypeStruct((B * L,), jnp.int32),
        ],
        mesh=mesh,
        scratch_types=[
            pltpu.VMEM((RPW * L + 16,), jnp.int32),
            pltpu.VMEM((RPW * L + 16,), jnp.int32),
            pltpu.VMEM((RPW * L + 16,), jnp.int32),
            pltpu.SemaphoreType.DMA,
        ],
    )
    return f(ids_flat)


def _tc_body(ids_hbm, pos_hbm, memb_hbm, out_hbm,
             buf0, buf1, buf2, buf3, idsv0, idsv1, labv0, labv1,
             rowzv, rowzs, membv,
             insems, outsems, idssems, rowzsem, membsem):
    bufs = (buf0, buf1, buf2, buf3)
    idsv = (idsv0, idsv1)
    labv = (labv0, labv1)

    pltpu.make_async_copy(memb_hbm, membv, membsem).start()
    pltpu.make_async_copy(memb_hbm, membv, membsem).wait()

    def in_cp(s, t):
        return pltpu.make_async_copy(
            pos_hbm.at[pl.ds(t * BBM, BBM)], bufs[s], insems.at[s])

    def out_cp(s, t):
        return pltpu.make_async_copy(
            bufs[s], out_hbm.at[pl.ds(t * BBM, BBM)], outsems.at[s])

    def ids_cp(p, t):
        return pltpu.make_async_copy(
            ids_hbm.at[pl.ds(t * BBM, BBM)], idsv[p], idssems.at[p])

    for s in range(NSLOT):
        in_cp(s, s).start()
    ids_cp(0, 0).start()
    ids_cp(1, 1).start()

    lane = jax.lax.broadcasted_iota(jnp.int32, (BBM, L), 1)
    memb = membv[...]

    def step(q, s):
        t = NSLOT * q + s
        p = s % 2
        ids_cp(p, t).wait()
        ids = idsv[p][...]
        labels = jnp.where(lane == (L - 1), 0, jnp.roll(ids, -1, axis=1))
        labv[p][...] = labels

        @pl.when(t + 2 < NSTEPS)
        def _ids_next():
            ids_cp(p, t + 2).start()

        in_cp(s, t).wait()
        buf = bufs[s]
        buf[:, L - 1, :] = jnp.broadcast_to(memb, (BBM, D))

        zero = jnp.logical_and(labels == 0, lane < (L - 1))
        anyz = jnp.any(zero)

        @pl.when(anyz)
        def _slow():
            rowzv[...] = jnp.any(zero, axis=1, keepdims=True).astype(jnp.int32)
            pltpu.make_async_copy(rowzv, rowzs, rowzsem).start()
            pltpu.make_async_copy(rowzv, rowzs, rowzsem).wait()

            def rbody(b, _):
                @pl.when(rowzs[b, 0] != 0)
                def _fix():
                    labrow = labv[p][pl.ds(b, 1), :]
                    lab3 = jnp.transpose(labrow.reshape(1, 1, L), (0, 2, 1))
                    buf[pl.ds(b, 1)] = jnp.where(
                        lab3 != 0, buf[pl.ds(b, 1)], memb[None])
                return 0
            lax.fori_loop(0, BBM, rbody, 0)

        out_cp(s, t).start()

        @pl.when(t + NSLOT < NSTEPS)
        def _refill():
            out_cp(s, t).wait()
            in_cp(s, t + NSLOT).start()

    def loop_body(q, _):
        for s in range(NSLOT):
            step(q, s)
        return 0

    lax.fori_loop(0, NSTEPS // NSLOT, loop_body, 0)
    for s in range(NSLOT):
        out_cp(s, NSTEPS - NSLOT + s).wait()


def _tc_out(pos_emb, itemid_seq, memb2):
    return pl.pallas_call(
        _tc_body,
        in_specs=[
            pl.BlockSpec(memory_space=pl.ANY),
            pl.BlockSpec(memory_space=pl.ANY),
            pl.BlockSpec(memory_space=pl.ANY),
        ],
        out_specs=[pl.BlockSpec(memory_space=pl.ANY)],
        out_shape=[jax.ShapeDtypeStruct((B, L, D), jnp.float32)],
        scratch_shapes=[
            pltpu.VMEM((BBM, L, D), jnp.float32),
            pltpu.VMEM((BBM, L, D), jnp.float32),
            pltpu.VMEM((BBM, L, D), jnp.float32),
            pltpu.VMEM((BBM, L, D), jnp.float32),
            pltpu.VMEM((BBM, L), jnp.int32),
            pltpu.VMEM((BBM, L), jnp.int32),
            pltpu.VMEM((BBM, L), jnp.int32),
            pltpu.VMEM((BBM, L), jnp.int32),
            pltpu.VMEM((BBM, 1), jnp.int32),
            pltpu.SMEM((BBM, 1), jnp.int32),
            pltpu.VMEM((1, D), jnp.float32),
            pltpu.SemaphoreType.DMA((NSLOT,)),
            pltpu.SemaphoreType.DMA((NSLOT,)),
            pltpu.SemaphoreType.DMA((2,)),
            pltpu.SemaphoreType.DMA,
            pltpu.SemaphoreType.DMA,
        ],
    )(itemid_seq, pos_emb, memb2)


def kernel(pos_emb, itemid_seq, training, masked_item_embedding):
    del training
    lab_flat, mask_flat = _sc_labels_mask(itemid_seq.reshape(-1))
    (out,) = _tc_out(pos_emb, itemid_seq, masked_item_embedding.reshape(1, D))
    labels = lab_flat.reshape(B, L)
    mask = mask_flat.reshape(B, L) != 0
    return (out, labels, mask)
